# baseline (speedup 1.0000x reference)
import jax
import jax.numpy as jnp
from jax.experimental import pallas as pl
from jax.experimental.pallas import tpu as pltpu

H = 16777216
NG = 64
STRIDE = 266305
IDX = [STRIDE * i for i in range(NG)]
W = 128
RT = H // W
NCH = 32
CHR = RT // NCH
CHE = CHR * W
NB = 8
K = 4
ORD = list(range(1, NCH)) + [0]
ACT = [[i for i in range(NG) if c * CHE <= IDX[i] < (c + 1) * CHE]
       for c in range(NCH)]


def _body(x_hbm, gw_ref, sc_ref, lc_ref, out_hbm, buf, scr, sems_i, sems_o, sems_i2, sems_o2):
    fac = gw_ref[...] * jnp.tanh(sc_ref[...])
    l64 = jax.lax.broadcasted_iota(jnp.int32, (1, NG), 1)
    lane = jax.lax.broadcasted_iota(jnp.int32, (1, W), 1)

    ic = [pltpu.make_async_copy(
            x_hbm.at[pl.ds(ORD[t] * CHR, CHR), :],
            buf.at[pl.ds((t % NB) * CHR, CHR), :],
            (sems_i if t % 2 == 0 else sems_i2).at[t % NB]) for t in range(NCH)]
    oc = [pltpu.make_async_copy(
            buf.at[pl.ds((t % NB) * CHR, CHR), :],
            out_hbm.at[pl.ds(ORD[t] * CHR, CHR), :],
            (sems_o if t % 2 == 0 else sems_o2).at[t % NB]) for t in range(NCH)]

    def patch(t):
        c, b = ORD[t], t % NB
        for i in ACT[c]:
            brow = b * CHR + IDX[i] // W - c * CHR
            col = IDX[i] % W
            v = buf[pl.ds(brow, 1), :]
            xval = jnp.sum(jnp.where(lane == col, v, 0.0))
            fi = jnp.sum(jnp.where(l64 == i, fac, 0.0))
            g = xval * fi
            scr[...] = jnp.where(l64 == i, g, scr[...])
            buf[pl.ds(brow, 1), :] = jnp.where(lane == col, g, v)
        if c == 0:
            gv = scr[...]
            mean = jnp.sum(gv) / NG
            var = jnp.sum((gv - mean) ** 2) / (NG - 1)
            sig = jax.lax.rem(var, jnp.float32(1e-4)) * 1e-12
            srow = b * CHR
            v0 = buf[pl.ds(srow, 1), :]
            mp = lc_ref[...] * (1.0 + sig * 1e9)
            t22 = jnp.dot(v0[:, :22], mp,
                          preferred_element_type=jnp.float32)
            buf[pl.ds(srow, 1), :] = jnp.concatenate(
                [t22, v0[:, 22:]], axis=1)

    for t in range(NCH):
        if t >= NB:
            oc[t - NB].wait()
        ic[t].start()
        j = t - K
        if j >= 0:
            ic[j].wait()
            patch(j)
            oc[j].start()
    for j in range(NCH - K, NCH):
        ic[j].wait()
        patch(j)
        oc[j].start()
    for j in range(NCH - NB, NCH):
        oc[j].wait()


def kernel(x, gate_weights, sacred_combinations, aether_gates, letter_combinations):
    del aether_gates
    x2 = x.reshape(RT, W)
    gw2 = gate_weights.reshape(1, NG)
    sc2 = sacred_combinations.reshape(1, NG)

    out = pl.pallas_call(
        _body,
        in_specs=[
            pl.BlockSpec(memory_space=pltpu.MemorySpace.HBM),
            pl.BlockSpec(memory_space=pltpu.MemorySpace.VMEM),
            pl.BlockSpec(memory_space=pltpu.MemorySpace.VMEM),
            pl.BlockSpec(memory_space=pltpu.MemorySpace.VMEM),
        ],
        out_specs=pl.BlockSpec(memory_space=pltpu.MemorySpace.HBM),
        out_shape=jax.ShapeDtypeStruct((RT, W), jnp.float32),
        scratch_shapes=[
            pltpu.VMEM((NB * CHR, W), jnp.float32),
            pltpu.VMEM((1, NG), jnp.float32),
            pltpu.SemaphoreType.DMA((NB,)),
            pltpu.SemaphoreType.DMA((NB,)),
            pltpu.SemaphoreType.DMA((NB,)),
            pltpu.SemaphoreType.DMA((NB,)),
        ],
    )(x2, gw2, sc2, letter_combinations)
    return out.reshape(H)


# speedup vs baseline: 1.0022x; 1.0022x over previous
import jax
import jax.numpy as jnp
from jax.experimental import pallas as pl
from jax.experimental.pallas import tpu as pltpu

H = 16777216
NG = 64
STRIDE = 266305
IDX = [STRIDE * i for i in range(NG)]
W = 128
RT = H // W
NCH = 32
CHR = RT // NCH
CHE = CHR * W
NB = 8
K = 4
ORD = list(range(1, NCH)) + [0]
ACT = [[i for i in range(NG) if c * CHE <= IDX[i] < (c + 1) * CHE]
       for c in range(NCH)]


def _body(x_hbm, gw_ref, sc_ref, lc_ref, out_hbm, buf, scr, sems_i, sems_o):
    fac = gw_ref[...] * jnp.tanh(sc_ref[...])
    l64 = jax.lax.broadcasted_iota(jnp.int32, (1, NG), 1)
    lane = jax.lax.broadcasted_iota(jnp.int32, (1, W), 1)

    ic = [pltpu.make_async_copy(
            x_hbm.at[pl.ds(ORD[t] * CHR, CHR), :],
            buf.at[pl.ds((t % NB) * CHR, CHR), :],
            sems_i.at[t % NB]) for t in range(NCH)]
    oc = [pltpu.make_async_copy(
            buf.at[pl.ds((t % NB) * CHR, CHR), :],
            out_hbm.at[pl.ds(ORD[t] * CHR, CHR), :],
            sems_o.at[t % NB]) for t in range(NCH)]

    def patch(t):
        c, b = ORD[t], t % NB
        for i in ACT[c]:
            brow = b * CHR + IDX[i] // W - c * CHR
            col = IDX[i] % W
            v = buf[pl.ds(brow, 1), :]
            xval = jnp.sum(jnp.where(lane == col, v, 0.0))
            fi = jnp.sum(jnp.where(l64 == i, fac, 0.0))
            g = xval * fi
            scr[...] = jnp.where(l64 == i, g, scr[...])
            buf[pl.ds(brow, 1), :] = jnp.where(lane == col, g, v)
        if c == 0:
            gv = scr[...]
            mean = jnp.sum(gv) / NG
            var = jnp.sum((gv - mean) ** 2) / (NG - 1)
            sig = jax.lax.rem(var, jnp.float32(1e-4)) * 1e-12
            srow = b * CHR
            v0 = buf[pl.ds(srow, 1), :]
            mp = lc_ref[...] * (1.0 + sig * 1e9)
            t22 = jnp.dot(v0[:, :22], mp,
                          preferred_element_type=jnp.float32)
            buf[pl.ds(srow, 1), :] = jnp.concatenate(
                [t22, v0[:, 22:]], axis=1)

    for t in range(NCH):
        if t >= NB:
            oc[t - NB].wait()
        ic[t].start()
        j = t - K
        if j >= 0:
            ic[j].wait()
            patch(j)
            oc[j].start()
    for j in range(NCH - K, NCH):
        ic[j].wait()
        patch(j)
        oc[j].start()
    for j in range(NCH - NB, NCH):
        oc[j].wait()


def kernel(x, gate_weights, sacred_combinations, aether_gates, letter_combinations):
    del aether_gates
    x2 = x.reshape(RT, W)
    gw2 = gate_weights.reshape(1, NG)
    sc2 = sacred_combinations.reshape(1, NG)

    out = pl.pallas_call(
        _body,
        in_specs=[
            pl.BlockSpec(memory_space=pltpu.MemorySpace.HBM),
            pl.BlockSpec(memory_space=pltpu.MemorySpace.VMEM),
            pl.BlockSpec(memory_space=pltpu.MemorySpace.VMEM),
            pl.BlockSpec(memory_space=pltpu.MemorySpace.VMEM),
        ],
        out_specs=pl.BlockSpec(memory_space=pltpu.MemorySpace.HBM),
        out_shape=jax.ShapeDtypeStruct((RT, W), jnp.float32),
        scratch_shapes=[
            pltpu.VMEM((NB * CHR, W), jnp.float32),
            pltpu.VMEM((1, NG), jnp.float32),
            pltpu.SemaphoreType.DMA((NB,)),
            pltpu.SemaphoreType.DMA((NB,)),
        ],
    )(x2, gw2, sc2, letter_combinations)
    return out.reshape(H)


# speedup vs baseline: 1.0033x; 1.0012x over previous
import jax
import jax.numpy as jnp
from jax.experimental import pallas as pl
from jax.experimental.pallas import tpu as pltpu

H = 16777216
NG = 64
STRIDE = 266305
IDX = [STRIDE * i for i in range(NG)]
W = 128
RT = H // W
NCH = 32
CHR = RT // NCH
CHE = CHR * W
NB = 8
K = 4
ORD = list(range(1, NCH)) + [0]
ACT = [[i for i in range(NG) if c * CHE <= IDX[i] < (c + 1) * CHE]
       for c in range(NCH)]


def _body(x_hbm, gw_ref, sc_ref, lc_ref, out_hbm, buf, scr, sems_i, sems_o):
    fac = gw_ref[...] * jnp.tanh(sc_ref[...])
    l64 = jax.lax.broadcasted_iota(jnp.int32, (1, NG), 1)
    lane = jax.lax.broadcasted_iota(jnp.int32, (1, W), 1)

    ic = [pltpu.make_async_copy(
            x_hbm.at[pl.ds(ORD[t] * CHR, CHR), :],
            buf.at[pl.ds((t % NB) * CHR, CHR), :],
            sems_i.at[t % NB]) for t in range(NCH)]
    oc = [pltpu.make_async_copy(
            buf.at[pl.ds((t % NB) * CHR, CHR), :],
            out_hbm.at[pl.ds(ORD[t] * CHR, CHR), :],
            sems_o.at[t % NB]) for t in range(NCH)]

    def patch(t):
        c, b = ORD[t], t % NB
        for i in ACT[c]:
            brow = b * CHR + IDX[i] // W - c * CHR
            col = IDX[i] % W
            fi = jnp.sum(jnp.where(l64 == i, fac, 0.0))
            g = buf[brow, col] * fi
            buf[pl.ds(brow, 1), :] = jnp.where(
                lane == col, g, buf[pl.ds(brow, 1), :])
            scr[...] = jnp.where(l64 == i, g, scr[...])
        if c == 0:
            gv = scr[...]
            mean = jnp.sum(gv) / NG
            var = jnp.sum((gv - mean) ** 2) / (NG - 1)
            sig = jax.lax.rem(var, jnp.float32(1e-4)) * 1e-12
            srow = b * CHR
            v0 = buf[pl.ds(srow, 1), :]
            mp = lc_ref[...] * (1.0 + sig * 1e9)
            t22 = jnp.dot(v0[:, :22], mp,
                          preferred_element_type=jnp.float32)
            buf[pl.ds(srow, 1), :] = jnp.concatenate(
                [t22, v0[:, 22:]], axis=1)

    for t in range(NCH):
        if t >= NB:
            oc[t - NB].wait()
        ic[t].start()
        j = t - K
        if j >= 0:
            ic[j].wait()
            patch(j)
            oc[j].start()
    for j in range(NCH - K, NCH):
        ic[j].wait()
        patch(j)
        oc[j].start()
    for j in range(NCH - NB, NCH):
        oc[j].wait()


def kernel(x, gate_weights, sacred_combinations, aether_gates, letter_combinations):
    del aether_gates
    x2 = x.reshape(RT, W)
    gw2 = gate_weights.reshape(1, NG)
    sc2 = sacred_combinations.reshape(1, NG)

    out = pl.pallas_call(
        _body,
        in_specs=[
            pl.BlockSpec(memory_space=pltpu.MemorySpace.HBM),
            pl.BlockSpec(memory_space=pltpu.MemorySpace.VMEM),
            pl.BlockSpec(memory_space=pltpu.MemorySpace.VMEM),
            pl.BlockSpec(memory_space=pltpu.MemorySpace.VMEM),
        ],
        out_specs=pl.BlockSpec(memory_space=pltpu.MemorySpace.HBM),
        out_shape=jax.ShapeDtypeStruct((RT, W), jnp.float32),
        scratch_shapes=[
            pltpu.VMEM((NB * CHR, W), jnp.float32),
            pltpu.VMEM((1, NG), jnp.float32),
            pltpu.SemaphoreType.DMA((NB,)),
            pltpu.SemaphoreType.DMA((NB,)),
        ],
    )(x2, gw2, sc2, letter_combinations)
    return out.reshape(H)
